# baseline (speedup 1.0000x reference)
import functools

import jax
import jax.numpy as jnp
import numpy as np
from jax import lax
from jax.experimental import pallas as pl
from jax.experimental.pallas import tpu as pltpu
from jax.experimental.pallas import tpu_sc as plsc

R = 64
C = 1_000_000
K = 24
W = 8192
NB = (C + W - 1) // W

_MINVAL = np.float32(1e-7)
_MAXVAL = np.float32(1.0 - 1e-7)
_SCALE = np.float32(_MAXVAL - _MINVAL)
_KS1 = 42
_KS2 = 42 ^ 0x1BD11BDA
_ROT = (13, 15, 26, 6, 17, 29, 16, 24)

_NTILES = 32
_PER_TILE = K * C // _NTILES
_SBUF = 50_000
_NBUF = _PER_TILE // _SBUF
_VPI = 5
_INNER = _SBUF // (16 * _VPI)


def _rotl(x, d):
    return lax.shift_left(x, jnp.int32(d)) | lax.shift_right_logical(
        x, jnp.int32(32 - d))


def _threefry_bits(flat):
    ks = (jnp.int32(0), jnp.int32(_KS1), jnp.int32(_KS2))
    x1 = flat + ks[1]
    x0 = x1
    x1 = _rotl(x1, _ROT[0]) ^ x0
    for d in _ROT[1:4]:
        x0 = x0 + x1
        x1 = _rotl(x1, d) ^ x0
    x0 = x0 + ks[1]
    x1 = x1 + ks[2] + jnp.int32(1)
    for g in range(1, 5):
        rots = _ROT[0:4] if g % 2 == 0 else _ROT[4:8]
        for d in rots:
            x0 = x0 + x1
            x1 = _rotl(x1, d) ^ x0
        x0 = x0 + ks[(g + 1) % 3]
        x1 = x1 + ks[(g + 2) % 3] + jnp.int32(g + 1)
    return x0 ^ x1


def _gumbel_from_bits(bits):
    fb = lax.shift_right_logical(bits, jnp.int32(9)) | jnp.int32(0x3F800000)
    fl = lax.bitcast_convert_type(fb, jnp.float32) - jnp.float32(1.0)
    u = jnp.maximum(_MINVAL, fl * _SCALE + _MINVAL)
    return -jnp.log(-jnp.log(u))


def _sc_body(out_ref, buf0, buf1, sem0, sem1):
    wid = lax.axis_index("c") * 16 + lax.axis_index("s")
    tbase = wid * _PER_TILE
    iota16 = lax.broadcasted_iota(jnp.int32, (16,), 0)

    def fill(bufref, b):
        start = tbase + b * _SBUF

        def fbody(it, _):
            off = it * (16 * _VPI)
            for vv in range(_VPI):
                o2 = off + vv * 16
                bufref[pl.ds(o2, 16)] = _threefry_bits(start + o2 + iota16)
            return 0

        lax.fori_loop(0, _INNER, fbody, 0)

    def copy(bufref, b, sem):
        return pltpu.make_async_copy(
            bufref, out_ref.at[pl.ds(tbase + b * _SBUF, _SBUF)], sem)

    def outer(h, _):
        for s, bufref, sem in ((0, buf0, sem0), (1, buf1, sem1)):
            b = 2 * h + s

            @pl.when(h > 0)
            def _():
                copy(bufref, b, sem).wait()

            fill(bufref, b)
            copy(bufref, b, sem).start()
        return 0

    lax.fori_loop(0, (_NBUF - 1) // 2, outer, 0)
    blast = _NBUF - 1
    copy(buf0, blast, sem0).wait()
    fill(buf0, blast)
    copy(buf0, blast, sem0).start()
    copy(buf0, blast, sem0).wait()
    copy(buf1, blast - 1, sem1).wait()


def _sc_bits():
    mesh = plsc.VectorSubcoreMesh(core_axis_name="c", subcore_axis_name="s")
    f = pl.kernel(
        _sc_body,
        mesh=mesh,
        out_type=jax.ShapeDtypeStruct((K * C,), jnp.int32),
        scratch_types=[
            pltpu.VMEM((_SBUF,), jnp.int32),
            pltpu.VMEM((_SBUF,), jnp.int32),
            pltpu.SemaphoreType.DMA,
            pltpu.SemaphoreType.DMA,
        ],
    )
    return f()


def _argmax_update(vs, cs, rm, ri):
    while len(vs) > 1:
        nvs, ncs = [], []
        for a in range(0, len(vs) - 1, 2):
            keep = vs[a] >= vs[a + 1]
            nvs.append(jnp.where(keep, vs[a], vs[a + 1]))
            ncs.append(jnp.where(keep, cs[a], cs[a + 1]))
        if len(vs) % 2:
            nvs.append(vs[-1])
            ncs.append(cs[-1])
        vs, cs = nvs, ncs
    take = vs[0] > rm
    return jnp.where(take, vs[0], rm), jnp.where(take, cs[0], ri)


def _finalize(rm, ri, o_ref, rm_ref, ri_ref, j):
    bm = jnp.max(rm, axis=1, keepdims=True)
    bi = jnp.min(jnp.where(rm == bm, ri, jnp.int32(0x7FFFFFFF)),
                 axis=1, keepdims=True)
    take = bm > rm_ref[:]
    rm_ref[:] = jnp.where(take, bm, rm_ref[:])
    ri_ref[:] = jnp.where(take, bi, ri_ref[:])

    @pl.when(j == NB - 1)
    def _():
        o_ref[:] = ri_ref[:]


_CH_A = 128
_CH_B = 1024


def _tc_hash_kernel(x_ref, o_ref, rm_ref, ri_ref):
    rg = pl.program_id(0)
    j = pl.program_id(1)

    @pl.when(j == 0)
    def _():
        rm_ref[:] = jnp.full((8, 1), -jnp.inf, jnp.float32)
        ri_ref[:] = jnp.zeros((8, 1), jnp.int32)

    base = j * W
    rowbase = K + rg * 8
    row = lax.broadcasted_iota(jnp.int32, (8, _CH_A), 0) + rowbase
    lane = lax.broadcasted_iota(jnp.int32, (8, _CH_A), 1)
    lane128 = lax.broadcasted_iota(jnp.int32, (8, 128), 1)
    nsub = _CH_A // 128

    def body(t, carry):
        rm, ri = carry
        off = pl.multiple_of(t * _CH_A, _CH_A)
        colbase = base + off
        g = _gumbel_from_bits(_threefry_bits(row * C + (colbase + lane)))
        v = x_ref[:, pl.ds(off, _CH_A)] + g
        vs = [v[:, k * 128:(k + 1) * 128] for k in range(nsub)]
        cs = [colbase + k * 128 + lane128 for k in range(nsub)]
        vs = [jnp.where(c < C, vv, -jnp.inf) for vv, c in zip(vs, cs)]
        return _argmax_update(vs, cs, rm, ri)

    rm0 = jnp.full((8, 128), -jnp.inf, jnp.float32)
    ri0 = jnp.zeros((8, 128), jnp.int32)
    rm, ri = lax.fori_loop(0, W // _CH_A, body, (rm0, ri0))
    _finalize(rm, ri, o_ref, rm_ref, ri_ref, j)


def _tc_bits_kernel(x_ref, b_ref, o_ref, rm_ref, ri_ref):
    j = pl.program_id(1)

    @pl.when(j == 0)
    def _():
        rm_ref[:] = jnp.full((8, 1), -jnp.inf, jnp.float32)
        ri_ref[:] = jnp.zeros((8, 1), jnp.int32)

    base = j * W
    lane128 = lax.broadcasted_iota(jnp.int32, (8, 128), 1)
    nsub = _CH_B // 128

    def body(t, carry):
        rm, ri = carry
        off = pl.multiple_of(t * _CH_B, _CH_B)
        colbase = base + off
        g = _gumbel_from_bits(b_ref[:, pl.ds(off, _CH_B)])
        v = x_ref[:, pl.ds(off, _CH_B)] + g
        vs = [v[:, k * 128:(k + 1) * 128] for k in range(nsub)]
        cs = [colbase + k * 128 + lane128 for k in range(nsub)]
        vs = [jnp.where(c < C, vv, -jnp.inf) for vv, c in zip(vs, cs)]
        return _argmax_update(vs, cs, rm, ri)

    rm0 = jnp.full((8, 128), -jnp.inf, jnp.float32)
    ri0 = jnp.zeros((8, 128), jnp.int32)
    rm, ri = lax.fori_loop(0, W // _CH_B, body, (rm0, ri0))
    _finalize(rm, ri, o_ref, rm_ref, ri_ref, j)


@jax.jit
def kernel(logits):
    bits = _sc_bits()

    out_a = pl.pallas_call(
        _tc_hash_kernel,
        grid=((R - K) // 8, NB),
        in_specs=[pl.BlockSpec((8, W), lambda rg, j: (K // 8 + rg, j))],
        out_specs=pl.BlockSpec((8, 1), lambda rg, j: (rg, 0)),
        out_shape=jax.ShapeDtypeStruct((R - K, 1), jnp.int32),
        scratch_shapes=[
            pltpu.VMEM((8, 1), jnp.float32),
            pltpu.VMEM((8, 1), jnp.int32),
        ],
    )(logits)

    out_b = pl.pallas_call(
        _tc_bits_kernel,
        grid=(K // 8, NB),
        in_specs=[
            pl.BlockSpec((8, W), lambda rg, j: (rg, j)),
            pl.BlockSpec((8, W), lambda rg, j: (rg, j)),
        ],
        out_specs=pl.BlockSpec((8, 1), lambda rg, j: (rg, 0)),
        out_shape=jax.ShapeDtypeStruct((K, 1), jnp.int32),
        scratch_shapes=[
            pltpu.VMEM((8, 1), jnp.float32),
            pltpu.VMEM((8, 1), jnp.int32),
        ],
    )(logits, bits.reshape(K, C))

    return jnp.concatenate([out_b.reshape(K), out_a.reshape(R - K)])


# speedup vs baseline: 1.8355x; 1.8355x over previous
import functools

import jax
import jax.numpy as jnp
import numpy as np
from jax import lax
from jax.experimental import pallas as pl
from jax.experimental.pallas import tpu as pltpu
from jax.experimental.pallas import tpu_sc as plsc

R = 64
C = 1_000_000
K = 24
W = 8192
NB = (C + W - 1) // W

_MINVAL = np.float32(1e-7)
_MAXVAL = np.float32(1.0 - 1e-7)
_SCALE = np.float32(_MAXVAL - _MINVAL)
_KS1 = 42
_KS2 = 42 ^ 0x1BD11BDA
_ROT = (13, 15, 26, 6, 17, 29, 16, 24)

_NTILES = 32
_PER_TILE = K * C // _NTILES
_SBUF = 50_000
_NBUF = _PER_TILE // _SBUF
_VPI = 25
_INNER = _SBUF // (16 * _VPI)


def _rotl(x, d):
    return lax.shift_left(x, jnp.int32(d)) | lax.shift_right_logical(
        x, jnp.int32(32 - d))


def _threefry_bits(flat):
    ks = (jnp.int32(0), jnp.int32(_KS1), jnp.int32(_KS2))
    x1 = flat + ks[1]
    x0 = x1
    x1 = _rotl(x1, _ROT[0]) ^ x0
    for d in _ROT[1:4]:
        x0 = x0 + x1
        x1 = _rotl(x1, d) ^ x0
    x0 = x0 + ks[1]
    x1 = x1 + ks[2] + jnp.int32(1)
    for g in range(1, 5):
        rots = _ROT[0:4] if g % 2 == 0 else _ROT[4:8]
        for d in rots:
            x0 = x0 + x1
            x1 = _rotl(x1, d) ^ x0
        x0 = x0 + ks[(g + 1) % 3]
        x1 = x1 + ks[(g + 2) % 3] + jnp.int32(g + 1)
    return x0 ^ x1


def _gumbel_from_bits(bits):
    fb = lax.shift_right_logical(bits, jnp.int32(9)) | jnp.int32(0x3F800000)
    fl = lax.bitcast_convert_type(fb, jnp.float32) - jnp.float32(1.0)
    u = jnp.maximum(_MINVAL, fl * _SCALE + _MINVAL)
    return -jnp.log(-jnp.log(u))


def _sc_body(out_ref, buf0, buf1, sem0, sem1):
    wid = lax.axis_index("c") * 16 + lax.axis_index("s")
    tbase = wid * _PER_TILE
    iota16 = lax.broadcasted_iota(jnp.int32, (16,), 0)

    def fill(bufref, b):
        start = tbase + b * _SBUF

        def fbody(it, _):
            off = it * (16 * _VPI)
            for vv in range(_VPI):
                o2 = off + vv * 16
                bufref[pl.ds(o2, 16)] = _threefry_bits(start + o2 + iota16)
            return 0

        lax.fori_loop(0, _INNER, fbody, 0)

    def copy(bufref, b, sem):
        return pltpu.make_async_copy(
            bufref, out_ref.at[pl.ds(tbase + b * _SBUF, _SBUF)], sem)

    def outer(h, _):
        for s, bufref, sem in ((0, buf0, sem0), (1, buf1, sem1)):
            b = 2 * h + s

            @pl.when(h > 0)
            def _():
                copy(bufref, b, sem).wait()

            fill(bufref, b)
            copy(bufref, b, sem).start()
        return 0

    lax.fori_loop(0, (_NBUF - 1) // 2, outer, 0)
    blast = _NBUF - 1
    copy(buf0, blast, sem0).wait()
    fill(buf0, blast)
    copy(buf0, blast, sem0).start()
    copy(buf0, blast, sem0).wait()
    copy(buf1, blast - 1, sem1).wait()


def _sc_bits():
    mesh = plsc.VectorSubcoreMesh(core_axis_name="c", subcore_axis_name="s")
    f = pl.kernel(
        _sc_body,
        mesh=mesh,
        out_type=jax.ShapeDtypeStruct((K * C,), jnp.int32),
        scratch_types=[
            pltpu.VMEM((_SBUF,), jnp.int32),
            pltpu.VMEM((_SBUF,), jnp.int32),
            pltpu.SemaphoreType.DMA,
            pltpu.SemaphoreType.DMA,
        ],
    )
    return f()


def _argmax_update(vs, cs, rm, ri):
    while len(vs) > 1:
        nvs, ncs = [], []
        for a in range(0, len(vs) - 1, 2):
            keep = vs[a] >= vs[a + 1]
            nvs.append(jnp.where(keep, vs[a], vs[a + 1]))
            ncs.append(jnp.where(keep, cs[a], cs[a + 1]))
        if len(vs) % 2:
            nvs.append(vs[-1])
            ncs.append(cs[-1])
        vs, cs = nvs, ncs
    take = vs[0] > rm
    return jnp.where(take, vs[0], rm), jnp.where(take, cs[0], ri)


def _finalize(rm, ri, o_ref, rm_ref, ri_ref, j):
    bm = jnp.max(rm, axis=1, keepdims=True)
    bi = jnp.min(jnp.where(rm == bm, ri, jnp.int32(0x7FFFFFFF)),
                 axis=1, keepdims=True)
    take = bm > rm_ref[:]
    rm_ref[:] = jnp.where(take, bm, rm_ref[:])
    ri_ref[:] = jnp.where(take, bi, ri_ref[:])

    @pl.when(j == NB - 1)
    def _():
        o_ref[:] = ri_ref[:]


_CH_A = 1024
_CH_B = 1024


def _tc_hash_kernel(x_ref, o_ref, rm_ref, ri_ref):
    rg = pl.program_id(0)
    j = pl.program_id(1)

    @pl.when(j == 0)
    def _():
        rm_ref[:] = jnp.full((8, 1), -jnp.inf, jnp.float32)
        ri_ref[:] = jnp.zeros((8, 1), jnp.int32)

    base = j * W
    rowbase = K + rg * 8
    row = lax.broadcasted_iota(jnp.int32, (8, _CH_A), 0) + rowbase
    lane = lax.broadcasted_iota(jnp.int32, (8, _CH_A), 1)
    lane128 = lax.broadcasted_iota(jnp.int32, (8, 128), 1)
    nsub = _CH_A // 128

    def body(t, carry):
        rm, ri = carry
        off = pl.multiple_of(t * _CH_A, _CH_A)
        colbase = base + off
        g = _gumbel_from_bits(_threefry_bits(row * C + (colbase + lane)))
        v = x_ref[:, pl.ds(off, _CH_A)] + g
        vs = [v[:, k * 128:(k + 1) * 128] for k in range(nsub)]
        cs = [colbase + k * 128 + lane128 for k in range(nsub)]
        vs = [jnp.where(c < C, vv, -jnp.inf) for vv, c in zip(vs, cs)]
        return _argmax_update(vs, cs, rm, ri)

    rm0 = jnp.full((8, 128), -jnp.inf, jnp.float32)
    ri0 = jnp.zeros((8, 128), jnp.int32)
    rm, ri = lax.fori_loop(0, W // _CH_A, body, (rm0, ri0))
    _finalize(rm, ri, o_ref, rm_ref, ri_ref, j)


def _tc_bits_kernel(x_ref, b_ref, o_ref, rm_ref, ri_ref):
    j = pl.program_id(1)

    @pl.when(j == 0)
    def _():
        rm_ref[:] = jnp.full((8, 1), -jnp.inf, jnp.float32)
        ri_ref[:] = jnp.zeros((8, 1), jnp.int32)

    base = j * W
    lane128 = lax.broadcasted_iota(jnp.int32, (8, 128), 1)
    nsub = _CH_B // 128

    def body(t, carry):
        rm, ri = carry
        off = pl.multiple_of(t * _CH_B, _CH_B)
        colbase = base + off
        g = _gumbel_from_bits(b_ref[:, pl.ds(off, _CH_B)])
        v = x_ref[:, pl.ds(off, _CH_B)] + g
        vs = [v[:, k * 128:(k + 1) * 128] for k in range(nsub)]
        cs = [colbase + k * 128 + lane128 for k in range(nsub)]
        vs = [jnp.where(c < C, vv, -jnp.inf) for vv, c in zip(vs, cs)]
        return _argmax_update(vs, cs, rm, ri)

    rm0 = jnp.full((8, 128), -jnp.inf, jnp.float32)
    ri0 = jnp.zeros((8, 128), jnp.int32)
    rm, ri = lax.fori_loop(0, W // _CH_B, body, (rm0, ri0))
    _finalize(rm, ri, o_ref, rm_ref, ri_ref, j)


@jax.jit
def kernel(logits):
    bits = _sc_bits()

    out_a = pl.pallas_call(
        _tc_hash_kernel,
        grid=((R - K) // 8, NB),
        in_specs=[pl.BlockSpec((8, W), lambda rg, j: (K // 8 + rg, j))],
        out_specs=pl.BlockSpec((8, 1), lambda rg, j: (rg, 0)),
        out_shape=jax.ShapeDtypeStruct((R - K, 1), jnp.int32),
        scratch_shapes=[
            pltpu.VMEM((8, 1), jnp.float32),
            pltpu.VMEM((8, 1), jnp.int32),
        ],
    )(logits)

    out_b = pl.pallas_call(
        _tc_bits_kernel,
        grid=(K // 8, NB),
        in_specs=[
            pl.BlockSpec((8, W), lambda rg, j: (rg, j)),
            pl.BlockSpec((8, W), lambda rg, j: (rg, j)),
        ],
        out_specs=pl.BlockSpec((8, 1), lambda rg, j: (rg, 0)),
        out_shape=jax.ShapeDtypeStruct((K, 1), jnp.int32),
        scratch_shapes=[
            pltpu.VMEM((8, 1), jnp.float32),
            pltpu.VMEM((8, 1), jnp.int32),
        ],
    )(logits, bits.reshape(K, C))

    return jnp.concatenate([out_b.reshape(K), out_a.reshape(R - K)])


# speedup vs baseline: 2.4263x; 1.3219x over previous
import functools

import jax
import jax.numpy as jnp
import numpy as np
from jax import lax
from jax.experimental import pallas as pl
from jax.experimental.pallas import tpu as pltpu
from jax.experimental.pallas import tpu_sc as plsc

R = 64
C = 1_000_000
KA = 48
KB = R - KA
W = 8192
NB = (C + W - 1) // W

_MINVAL = np.float32(1e-7)
_MAXVAL = np.float32(1.0 - 1e-7)
_SCALE = np.float32(_MAXVAL - _MINVAL)
_KS1 = 42
_KS2 = 42 ^ 0x1BD11BDA
_ROT = (13, 15, 26, 6, 17, 29, 16, 24)

_NTILES = 32
_PER_TILE = KB * C // _NTILES
_SBUF = 50_000
_NBUF = _PER_TILE // _SBUF
_VPI = 25
_INNER = _SBUF // (16 * _VPI)


def _rotl(x, d):
    return lax.shift_left(x, jnp.int32(d)) | lax.shift_right_logical(
        x, jnp.int32(32 - d))


def _threefry_bits(flat):
    ks = (jnp.int32(0), jnp.int32(_KS1), jnp.int32(_KS2))
    x1 = flat + ks[1]
    x0 = x1
    x1 = _rotl(x1, _ROT[0]) ^ x0
    for d in _ROT[1:4]:
        x0 = x0 + x1
        x1 = _rotl(x1, d) ^ x0
    x0 = x0 + ks[1]
    x1 = x1 + ks[2] + jnp.int32(1)
    for g in range(1, 5):
        rots = _ROT[0:4] if g % 2 == 0 else _ROT[4:8]
        for d in rots:
            x0 = x0 + x1
            x1 = _rotl(x1, d) ^ x0
        x0 = x0 + ks[(g + 1) % 3]
        x1 = x1 + ks[(g + 2) % 3] + jnp.int32(g + 1)
    return x0 ^ x1


def _gumbel_from_bits(bits):
    fb = lax.shift_right_logical(bits, jnp.int32(9)) | jnp.int32(0x3F800000)
    fl = lax.bitcast_convert_type(fb, jnp.float32) - jnp.float32(1.0)
    u = jnp.maximum(_MINVAL, fl * _SCALE + _MINVAL)
    return -jnp.log(-jnp.log(u))


def _sc_body(out_ref, buf0, buf1, sem0, sem1):
    wid = lax.axis_index("c") * 16 + lax.axis_index("s")
    tbase = wid * _PER_TILE
    iota16 = lax.broadcasted_iota(jnp.int32, (16,), 0)

    def fill(bufref, b):
        start = KA * C + tbase + b * _SBUF

        def fbody(it, _):
            off = it * (16 * _VPI)
            for vv in range(_VPI):
                o2 = off + vv * 16
                bufref[pl.ds(o2, 16)] = _threefry_bits(start + o2 + iota16)
            return 0

        lax.fori_loop(0, _INNER, fbody, 0)

    def copy(bufref, b, sem):
        return pltpu.make_async_copy(
            bufref, out_ref.at[pl.ds(tbase + b * _SBUF, _SBUF)], sem)

    def outer(h, _):
        for s, bufref, sem in ((0, buf0, sem0), (1, buf1, sem1)):
            b = 2 * h + s

            @pl.when(h > 0)
            def _():
                copy(bufref, b, sem).wait()

            fill(bufref, b)
            copy(bufref, b, sem).start()
        return 0

    lax.fori_loop(0, _NBUF // 2, outer, 0)
    copy(buf0, _NBUF - 2, sem0).wait()
    copy(buf1, _NBUF - 1, sem1).wait()


def _sc_bits():
    mesh = plsc.VectorSubcoreMesh(core_axis_name="c", subcore_axis_name="s")
    f = pl.kernel(
        _sc_body,
        mesh=mesh,
        out_type=jax.ShapeDtypeStruct((KB * C,), jnp.int32),
        scratch_types=[
            pltpu.VMEM((_SBUF,), jnp.int32),
            pltpu.VMEM((_SBUF,), jnp.int32),
            pltpu.SemaphoreType.DMA,
            pltpu.SemaphoreType.DMA,
        ],
    )
    return f()


def _argmax_update(vs, cs, rm, ri):
    while len(vs) > 1:
        nvs, ncs = [], []
        for a in range(0, len(vs) - 1, 2):
            keep = vs[a] >= vs[a + 1]
            nvs.append(jnp.where(keep, vs[a], vs[a + 1]))
            ncs.append(jnp.where(keep, cs[a], cs[a + 1]))
        if len(vs) % 2:
            nvs.append(vs[-1])
            ncs.append(cs[-1])
        vs, cs = nvs, ncs
    take = vs[0] > rm
    return jnp.where(take, vs[0], rm), jnp.where(take, cs[0], ri)


def _finalize(rows, rm, ri, o_ref, rm_ref, ri_ref, j):
    bm = jnp.max(rm, axis=1, keepdims=True)
    bi = jnp.min(jnp.where(rm == bm, ri, jnp.int32(0x7FFFFFFF)),
                 axis=1, keepdims=True)
    take = bm > rm_ref[:]
    rm_ref[:] = jnp.where(take, bm, rm_ref[:])
    ri_ref[:] = jnp.where(take, bi, ri_ref[:])

    @pl.when(j == NB - 1)
    def _():
        o_ref[:] = ri_ref[:]


_CH_A = 128
_CH_B = 512


def _tc_hash_kernel(x_ref, o_ref, rm_ref, ri_ref):
    j = pl.program_id(0)

    @pl.when(j == 0)
    def _():
        rm_ref[:] = jnp.full((KA, 1), -jnp.inf, jnp.float32)
        ri_ref[:] = jnp.zeros((KA, 1), jnp.int32)

    base = j * W
    row = lax.broadcasted_iota(jnp.int32, (KA, _CH_A), 0)
    lane = lax.broadcasted_iota(jnp.int32, (KA, _CH_A), 1)
    lane128 = lax.broadcasted_iota(jnp.int32, (KA, 128), 1)
    nsub = _CH_A // 128

    def body(t, carry):
        rm, ri = carry
        off = pl.multiple_of(t * _CH_A, _CH_A)
        colbase = base + off
        g = _gumbel_from_bits(_threefry_bits(row * C + (colbase + lane)))
        v = x_ref[:, pl.ds(off, _CH_A)] + g
        vs = [v[:, k * 128:(k + 1) * 128] for k in range(nsub)]
        cs = [colbase + k * 128 + lane128 for k in range(nsub)]
        vs = [jnp.where(c < C, vv, -jnp.inf) for vv, c in zip(vs, cs)]
        return _argmax_update(vs, cs, rm, ri)

    rm0 = jnp.full((KA, 128), -jnp.inf, jnp.float32)
    ri0 = jnp.zeros((KA, 128), jnp.int32)
    rm, ri = lax.fori_loop(0, W // _CH_A, body, (rm0, ri0))
    _finalize(KA, rm, ri, o_ref, rm_ref, ri_ref, j)


def _tc_bits_kernel(x_ref, b_ref, o_ref, rm_ref, ri_ref):
    j = pl.program_id(0)

    @pl.when(j == 0)
    def _():
        rm_ref[:] = jnp.full((KB, 1), -jnp.inf, jnp.float32)
        ri_ref[:] = jnp.zeros((KB, 1), jnp.int32)

    base = j * W
    lane128 = lax.broadcasted_iota(jnp.int32, (KB, 128), 1)
    nsub = _CH_B // 128

    def body(t, carry):
        rm, ri = carry
        off = pl.multiple_of(t * _CH_B, _CH_B)
        colbase = base + off
        g = _gumbel_from_bits(b_ref[:, pl.ds(off, _CH_B)])
        v = x_ref[:, pl.ds(off, _CH_B)] + g
        vs = [v[:, k * 128:(k + 1) * 128] for k in range(nsub)]
        cs = [colbase + k * 128 + lane128 for k in range(nsub)]
        vs = [jnp.where(c < C, vv, -jnp.inf) for vv, c in zip(vs, cs)]
        return _argmax_update(vs, cs, rm, ri)

    rm0 = jnp.full((KB, 128), -jnp.inf, jnp.float32)
    ri0 = jnp.zeros((KB, 128), jnp.int32)
    rm, ri = lax.fori_loop(0, W // _CH_B, body, (rm0, ri0))
    _finalize(KB, rm, ri, o_ref, rm_ref, ri_ref, j)


@jax.jit
def kernel(logits):
    bits = _sc_bits()

    out_a = pl.pallas_call(
        _tc_hash_kernel,
        grid=(NB,),
        in_specs=[pl.BlockSpec((KA, W), lambda j: (0, j))],
        out_specs=pl.BlockSpec((KA, 1), lambda j: (0, 0)),
        out_shape=jax.ShapeDtypeStruct((KA, 1), jnp.int32),
        scratch_shapes=[
            pltpu.VMEM((KA, 1), jnp.float32),
            pltpu.VMEM((KA, 1), jnp.int32),
        ],
    )(logits)

    out_b = pl.pallas_call(
        _tc_bits_kernel,
        grid=(NB,),
        in_specs=[
            pl.BlockSpec((KB, W), lambda j: (KA // KB, j)),
            pl.BlockSpec((KB, W), lambda j: (0, j)),
        ],
        out_specs=pl.BlockSpec((KB, 1), lambda j: (0, 0)),
        out_shape=jax.ShapeDtypeStruct((KB, 1), jnp.int32),
        scratch_shapes=[
            pltpu.VMEM((KB, 1), jnp.float32),
            pltpu.VMEM((KB, 1), jnp.int32),
        ],
    )(logits, bits.reshape(KB, C))

    return jnp.concatenate([out_a.reshape(KA), out_b.reshape(KB)])


# speedup vs baseline: 5.4253x; 2.2360x over previous
import functools

import jax
import jax.numpy as jnp
import numpy as np
from jax import lax
from jax.experimental import pallas as pl
from jax.experimental.pallas import tpu as pltpu

R = 64
C = 1_000_000
W = 8192
CHUNK = 128
NB = (C + W - 1) // W

_MINVAL = np.float32(1e-7)
_MAXVAL = np.float32(1.0 - 1e-7)
_SCALE = np.float32(_MAXVAL - _MINVAL)
_KS0 = 0
_KS1 = 42
_KS2 = 42 ^ 0x1BD11BDA
_ROT = (13, 15, 26, 6, 17, 29, 16, 24)


def _rotl(x, d):
    return lax.shift_left(x, jnp.int32(d)) | lax.shift_right_logical(
        x, jnp.int32(32 - d))


def _threefry_bits(flat):
    ks = (jnp.int32(_KS0), jnp.int32(_KS1), jnp.int32(_KS2))
    x1 = flat + ks[1]
    x0 = x1
    x1 = _rotl(x1, _ROT[0]) ^ x0
    for d in _ROT[1:4]:
        x0 = x0 + x1
        x1 = _rotl(x1, d) ^ x0
    x0 = x0 + ks[1]
    x1 = x1 + ks[2] + jnp.int32(1)
    for g in range(1, 5):
        rots = _ROT[0:4] if g % 2 == 0 else _ROT[4:8]
        for d in rots:
            x0 = x0 + x1
            x1 = _rotl(x1, d) ^ x0
        x0 = x0 + ks[(g + 1) % 3]
        x1 = x1 + ks[(g + 2) % 3] + jnp.int32(g + 1)
    return x0 ^ x1


def _gumbel_from_flat(flat):
    bits = _threefry_bits(flat)
    fb = lax.shift_right_logical(bits, jnp.int32(9)) | jnp.int32(0x3F800000)
    fl = lax.bitcast_convert_type(fb, jnp.float32) - jnp.float32(1.0)
    u = jnp.maximum(_MINVAL, fl * _SCALE + _MINVAL)
    return -jnp.log(-jnp.log(u))


def _kernel(x_ref, o_ref, rm_ref, ri_ref):
    j = pl.program_id(0)

    @pl.when(j == 0)
    def _init():
        rm_ref[:] = jnp.full((R, 1), -jnp.inf, jnp.float32)
        ri_ref[:] = jnp.zeros((R, 1), jnp.int32)

    base = j * W
    nsub = CHUNK // 128
    row = lax.broadcasted_iota(jnp.int32, (R, CHUNK), 0)
    lane = lax.broadcasted_iota(jnp.int32, (R, CHUNK), 1)
    lane128 = lax.broadcasted_iota(jnp.int32, (R, 128), 1)

    def body(t, carry):
        rm, ri = carry
        off = pl.multiple_of(t * CHUNK, CHUNK)
        colbase = base + off
        flat = row * C + (colbase + lane)
        g = _gumbel_from_flat(flat)
        v = x_ref[:, pl.ds(off, CHUNK)] + g
        vs = [v[:, k * 128:(k + 1) * 128] for k in range(nsub)]
        cs = [colbase + k * 128 + lane128 for k in range(nsub)]
        vs = [jnp.where(c < C, vv, -jnp.inf) for vv, c in zip(vs, cs)]
        while len(vs) > 1:
            nvs, ncs = [], []
            for a in range(0, len(vs) - 1, 2):
                keep = vs[a] >= vs[a + 1]
                nvs.append(jnp.where(keep, vs[a], vs[a + 1]))
                ncs.append(jnp.where(keep, cs[a], cs[a + 1]))
            if len(vs) % 2:
                nvs.append(vs[-1])
                ncs.append(cs[-1])
            vs, cs = nvs, ncs
        take = vs[0] > rm
        rm = jnp.where(take, vs[0], rm)
        ri = jnp.where(take, cs[0], ri)
        return rm, ri

    rm0 = jnp.full((R, 128), -jnp.inf, jnp.float32)
    ri0 = jnp.zeros((R, 128), jnp.int32)
    rm, ri = lax.fori_loop(0, W // CHUNK, body, (rm0, ri0))

    bm = jnp.max(rm, axis=1, keepdims=True)
    bi = jnp.min(jnp.where(rm == bm, ri, jnp.int32(0x7FFFFFFF)),
                 axis=1, keepdims=True)

    take = bm > rm_ref[:]
    rm_ref[:] = jnp.where(take, bm, rm_ref[:])
    ri_ref[:] = jnp.where(take, bi, ri_ref[:])

    @pl.when(j == NB - 1)
    def _fin():
        o_ref[:] = ri_ref[:]


@jax.jit
def kernel(logits):
    out = pl.pallas_call(
        _kernel,
        grid=(NB,),
        in_specs=[pl.BlockSpec((R, W), lambda j: (0, j))],
        out_specs=pl.BlockSpec((R, 1), lambda j: (0, 0)),
        out_shape=jax.ShapeDtypeStruct((R, 1), jnp.int32),
        scratch_shapes=[
            pltpu.VMEM((R, 1), jnp.float32),
            pltpu.VMEM((R, 1), jnp.int32),
        ],
    )(logits)
    return out.reshape(R)


# speedup vs baseline: 5.6537x; 1.0421x over previous
import functools

import jax
import jax.numpy as jnp
import numpy as np
from jax import lax
from jax.experimental import pallas as pl
from jax.experimental.pallas import tpu as pltpu

R = 64
C = 1_000_000
W = 8192
CHUNK = 256
NB = (C + W - 1) // W

_MINVAL = np.float32(1e-7)
_MAXVAL = np.float32(1.0 - 1e-7)
_SCALE = np.float32(_MAXVAL - _MINVAL)
_KS0 = 0
_KS1 = 42
_KS2 = 42 ^ 0x1BD11BDA
_ROT = (13, 15, 26, 6, 17, 29, 16, 24)


def _rotl(x, d):
    return lax.shift_left(x, jnp.int32(d)) | lax.shift_right_logical(
        x, jnp.int32(32 - d))


def _threefry_bits(flat):
    ks = (jnp.int32(_KS0), jnp.int32(_KS1), jnp.int32(_KS2))
    x1 = flat + ks[1]
    x0 = x1
    x1 = _rotl(x1, _ROT[0]) ^ x0
    for d in _ROT[1:4]:
        x0 = x0 + x1
        x1 = _rotl(x1, d) ^ x0
    x0 = x0 + ks[1]
    x1 = x1 + ks[2] + jnp.int32(1)
    for g in range(1, 5):
        rots = _ROT[0:4] if g % 2 == 0 else _ROT[4:8]
        for d in rots:
            x0 = x0 + x1
            x1 = _rotl(x1, d) ^ x0
        x0 = x0 + ks[(g + 1) % 3]
        x1 = x1 + ks[(g + 2) % 3] + jnp.int32(g + 1)
    return x0 ^ x1


def _gumbel_from_flat(flat):
    bits = _threefry_bits(flat)
    fb = lax.shift_right_logical(bits, jnp.int32(9)) | jnp.int32(0x3F800000)
    fl = lax.bitcast_convert_type(fb, jnp.float32) - jnp.float32(1.0)
    u = jnp.maximum(_MINVAL, fl * _SCALE + _MINVAL)
    return -jnp.log(-jnp.log(u))


def _kernel(x_ref, o_ref, rm_ref, ri_ref):
    j = pl.program_id(0)

    @pl.when(j == 0)
    def _init():
        rm_ref[:] = jnp.full((R, 1), -jnp.inf, jnp.float32)
        ri_ref[:] = jnp.zeros((R, 1), jnp.int32)

    base = j * W
    nsub = CHUNK // 128
    row = lax.broadcasted_iota(jnp.int32, (R, CHUNK), 0)
    lane = lax.broadcasted_iota(jnp.int32, (R, CHUNK), 1)
    lane128 = lax.broadcasted_iota(jnp.int32, (R, 128), 1)

    def body(t, carry):
        rm, ri = carry
        off = pl.multiple_of(t * CHUNK, CHUNK)
        colbase = base + off
        flat = row * C + (colbase + lane)
        g = _gumbel_from_flat(flat)
        v = x_ref[:, pl.ds(off, CHUNK)] + g
        vs = [v[:, k * 128:(k + 1) * 128] for k in range(nsub)]
        cs = [colbase + k * 128 + lane128 for k in range(nsub)]
        vs = [jnp.where(c < C, vv, -jnp.inf) for vv, c in zip(vs, cs)]
        while len(vs) > 1:
            nvs, ncs = [], []
            for a in range(0, len(vs) - 1, 2):
                keep = vs[a] >= vs[a + 1]
                nvs.append(jnp.where(keep, vs[a], vs[a + 1]))
                ncs.append(jnp.where(keep, cs[a], cs[a + 1]))
            if len(vs) % 2:
                nvs.append(vs[-1])
                ncs.append(cs[-1])
            vs, cs = nvs, ncs
        take = vs[0] > rm
        rm = jnp.where(take, vs[0], rm)
        ri = jnp.where(take, cs[0], ri)
        return rm, ri

    rm0 = jnp.full((R, 128), -jnp.inf, jnp.float32)
    ri0 = jnp.zeros((R, 128), jnp.int32)
    rm, ri = lax.fori_loop(0, W // CHUNK, body, (rm0, ri0))

    bm = jnp.max(rm, axis=1, keepdims=True)
    bi = jnp.min(jnp.where(rm == bm, ri, jnp.int32(0x7FFFFFFF)),
                 axis=1, keepdims=True)

    take = bm > rm_ref[:]
    rm_ref[:] = jnp.where(take, bm, rm_ref[:])
    ri_ref[:] = jnp.where(take, bi, ri_ref[:])

    @pl.when(j == NB - 1)
    def _fin():
        o_ref[:] = ri_ref[:]


@jax.jit
def kernel(logits):
    out = pl.pallas_call(
        _kernel,
        grid=(NB,),
        in_specs=[pl.BlockSpec((R, W), lambda j: (0, j))],
        out_specs=pl.BlockSpec((R, 1), lambda j: (0, 0)),
        out_shape=jax.ShapeDtypeStruct((R, 1), jnp.int32),
        scratch_shapes=[
            pltpu.VMEM((R, 1), jnp.float32),
            pltpu.VMEM((R, 1), jnp.int32),
        ],
    )(logits)
    return out.reshape(R)


# speedup vs baseline: 5.9395x; 1.0505x over previous
import functools

import jax
import jax.numpy as jnp
import numpy as np
from jax import lax
from jax.experimental import pallas as pl
from jax.experimental.pallas import tpu as pltpu

R = 64
C = 1_000_000
W = 7680
CHUNK = 384
NB = (C + W - 1) // W

_MINVAL = np.float32(1e-7)
_MAXVAL = np.float32(1.0 - 1e-7)
_SCALE = np.float32(_MAXVAL - _MINVAL)
_KS0 = 0
_KS1 = 42
_KS2 = 42 ^ 0x1BD11BDA
_ROT = (13, 15, 26, 6, 17, 29, 16, 24)


def _rotl(x, d):
    return lax.shift_left(x, jnp.int32(d)) | lax.shift_right_logical(
        x, jnp.int32(32 - d))


def _threefry_bits(flat):
    ks = (jnp.int32(_KS0), jnp.int32(_KS1), jnp.int32(_KS2))
    x1 = flat + ks[1]
    x0 = x1
    x1 = _rotl(x1, _ROT[0]) ^ x0
    for d in _ROT[1:4]:
        x0 = x0 + x1
        x1 = _rotl(x1, d) ^ x0
    x0 = x0 + ks[1]
    x1 = x1 + ks[2] + jnp.int32(1)
    for g in range(1, 5):
        rots = _ROT[0:4] if g % 2 == 0 else _ROT[4:8]
        for d in rots:
            x0 = x0 + x1
            x1 = _rotl(x1, d) ^ x0
        x0 = x0 + ks[(g + 1) % 3]
        x1 = x1 + ks[(g + 2) % 3] + jnp.int32(g + 1)
    return x0 ^ x1


def _gumbel_from_flat(flat):
    bits = _threefry_bits(flat)
    fb = lax.shift_right_logical(bits, jnp.int32(9)) | jnp.int32(0x3F800000)
    fl = lax.bitcast_convert_type(fb, jnp.float32) - jnp.float32(1.0)
    u = jnp.maximum(_MINVAL, fl * _SCALE + _MINVAL)
    return -jnp.log(-jnp.log(u))


def _kernel(x_ref, o_ref, rm_ref, ri_ref):
    j = pl.program_id(0)

    @pl.when(j == 0)
    def _init():
        rm_ref[:] = jnp.full((R, 1), -jnp.inf, jnp.float32)
        ri_ref[:] = jnp.zeros((R, 1), jnp.int32)

    base = j * W
    nsub = CHUNK // 128
    row = lax.broadcasted_iota(jnp.int32, (R, CHUNK), 0)
    lane = lax.broadcasted_iota(jnp.int32, (R, CHUNK), 1)
    lane128 = lax.broadcasted_iota(jnp.int32, (R, 128), 1)

    def body(t, carry):
        rm, ri = carry
        off = pl.multiple_of(t * CHUNK, CHUNK)
        colbase = base + off
        flat = row * C + (colbase + lane)
        g = _gumbel_from_flat(flat)
        v = x_ref[:, pl.ds(off, CHUNK)] + g
        vs = [v[:, k * 128:(k + 1) * 128] for k in range(nsub)]
        cs = [colbase + k * 128 + lane128 for k in range(nsub)]
        vs = [jnp.where(c < C, vv, -jnp.inf) for vv, c in zip(vs, cs)]
        while len(vs) > 1:
            nvs, ncs = [], []
            for a in range(0, len(vs) - 1, 2):
                keep = vs[a] >= vs[a + 1]
                nvs.append(jnp.where(keep, vs[a], vs[a + 1]))
                ncs.append(jnp.where(keep, cs[a], cs[a + 1]))
            if len(vs) % 2:
                nvs.append(vs[-1])
                ncs.append(cs[-1])
            vs, cs = nvs, ncs
        take = vs[0] > rm
        rm = jnp.where(take, vs[0], rm)
        ri = jnp.where(take, cs[0], ri)
        return rm, ri

    rm0 = jnp.full((R, 128), -jnp.inf, jnp.float32)
    ri0 = jnp.zeros((R, 128), jnp.int32)
    rm, ri = lax.fori_loop(0, W // CHUNK, body, (rm0, ri0))

    bm = jnp.max(rm, axis=1, keepdims=True)
    bi = jnp.min(jnp.where(rm == bm, ri, jnp.int32(0x7FFFFFFF)),
                 axis=1, keepdims=True)

    take = bm > rm_ref[:]
    rm_ref[:] = jnp.where(take, bm, rm_ref[:])
    ri_ref[:] = jnp.where(take, bi, ri_ref[:])

    @pl.when(j == NB - 1)
    def _fin():
        o_ref[:] = ri_ref[:]


@jax.jit
def kernel(logits):
    out = pl.pallas_call(
        _kernel,
        grid=(NB,),
        in_specs=[pl.BlockSpec((R, W), lambda j: (0, j))],
        out_specs=pl.BlockSpec((R, 1), lambda j: (0, 0)),
        out_shape=jax.ShapeDtypeStruct((R, 1), jnp.int32),
        scratch_shapes=[
            pltpu.VMEM((R, 1), jnp.float32),
            pltpu.VMEM((R, 1), jnp.int32),
        ],
    )(logits)
    return out.reshape(R)


# speedup vs baseline: 5.9976x; 1.0098x over previous
import functools

import jax
import jax.numpy as jnp
import numpy as np
from jax import lax
from jax.experimental import pallas as pl
from jax.experimental.pallas import tpu as pltpu

R = 64
C = 1_000_000
W = 12288
CHUNK = 384
NB = (C + W - 1) // W

_MINVAL = np.float32(1e-7)
_MAXVAL = np.float32(1.0 - 1e-7)
_SCALE = np.float32(_MAXVAL - _MINVAL)
_KS0 = 0
_KS1 = 42
_KS2 = 42 ^ 0x1BD11BDA
_ROT = (13, 15, 26, 6, 17, 29, 16, 24)


def _rotl(x, d):
    return lax.shift_left(x, jnp.int32(d)) | lax.shift_right_logical(
        x, jnp.int32(32 - d))


def _threefry_bits(flat):
    ks = (jnp.int32(_KS0), jnp.int32(_KS1), jnp.int32(_KS2))
    x1 = flat + ks[1]
    x0 = x1
    x1 = _rotl(x1, _ROT[0]) ^ x0
    for d in _ROT[1:4]:
        x0 = x0 + x1
        x1 = _rotl(x1, d) ^ x0
    x0 = x0 + ks[1]
    x1 = x1 + ks[2] + jnp.int32(1)
    for g in range(1, 5):
        rots = _ROT[0:4] if g % 2 == 0 else _ROT[4:8]
        for d in rots:
            x0 = x0 + x1
            x1 = _rotl(x1, d) ^ x0
        x0 = x0 + ks[(g + 1) % 3]
        x1 = x1 + ks[(g + 2) % 3] + jnp.int32(g + 1)
    return x0 ^ x1


def _gumbel_from_flat(flat):
    bits = _threefry_bits(flat)
    fb = lax.shift_right_logical(bits, jnp.int32(9)) | jnp.int32(0x3F800000)
    fl = lax.bitcast_convert_type(fb, jnp.float32) - jnp.float32(1.0)
    u = jnp.maximum(_MINVAL, fl * _SCALE + _MINVAL)
    return -jnp.log(-jnp.log(u))


def _kernel(x_ref, o_ref, rm_ref, ri_ref):
    j = pl.program_id(0)

    @pl.when(j == 0)
    def _init():
        rm_ref[:] = jnp.full((R, 1), -jnp.inf, jnp.float32)
        ri_ref[:] = jnp.zeros((R, 1), jnp.int32)

    base = j * W
    nsub = CHUNK // 128
    row = lax.broadcasted_iota(jnp.int32, (R, CHUNK), 0)
    lane = lax.broadcasted_iota(jnp.int32, (R, CHUNK), 1)
    lane128 = lax.broadcasted_iota(jnp.int32, (R, 128), 1)

    def body(t, carry):
        rm, ri = carry
        off = pl.multiple_of(t * CHUNK, CHUNK)
        colbase = base + off
        flat = row * C + (colbase + lane)
        g = _gumbel_from_flat(flat)
        v = x_ref[:, pl.ds(off, CHUNK)] + g
        vs = [v[:, k * 128:(k + 1) * 128] for k in range(nsub)]
        cs = [colbase + k * 128 + lane128 for k in range(nsub)]
        vs = [jnp.where(c < C, vv, -jnp.inf) for vv, c in zip(vs, cs)]
        while len(vs) > 1:
            nvs, ncs = [], []
            for a in range(0, len(vs) - 1, 2):
                keep = vs[a] >= vs[a + 1]
                nvs.append(jnp.where(keep, vs[a], vs[a + 1]))
                ncs.append(jnp.where(keep, cs[a], cs[a + 1]))
            if len(vs) % 2:
                nvs.append(vs[-1])
                ncs.append(cs[-1])
            vs, cs = nvs, ncs
        take = vs[0] > rm
        rm = jnp.where(take, vs[0], rm)
        ri = jnp.where(take, cs[0], ri)
        return rm, ri

    rm0 = jnp.full((R, 128), -jnp.inf, jnp.float32)
    ri0 = jnp.zeros((R, 128), jnp.int32)
    rm, ri = lax.fori_loop(0, W // CHUNK, body, (rm0, ri0))

    bm = jnp.max(rm, axis=1, keepdims=True)
    bi = jnp.min(jnp.where(rm == bm, ri, jnp.int32(0x7FFFFFFF)),
                 axis=1, keepdims=True)

    take = bm > rm_ref[:]
    rm_ref[:] = jnp.where(take, bm, rm_ref[:])
    ri_ref[:] = jnp.where(take, bi, ri_ref[:])

    @pl.when(j == NB - 1)
    def _fin():
        o_ref[:] = ri_ref[:]


@jax.jit
def kernel(logits):
    out = pl.pallas_call(
        _kernel,
        grid=(NB,),
        in_specs=[pl.BlockSpec((R, W), lambda j: (0, j))],
        out_specs=pl.BlockSpec((R, 1), lambda j: (0, 0)),
        out_shape=jax.ShapeDtypeStruct((R, 1), jnp.int32),
        scratch_shapes=[
            pltpu.VMEM((R, 1), jnp.float32),
            pltpu.VMEM((R, 1), jnp.int32),
        ],
    )(logits)
    return out.reshape(R)


# speedup vs baseline: 6.0585x; 1.0102x over previous
import functools

import jax
import jax.numpy as jnp
import numpy as np
from jax import lax
from jax.experimental import pallas as pl
from jax.experimental.pallas import tpu as pltpu

R = 64
C = 1_000_000
W = 12288
CHUNK = 384
NB = (C + W - 1) // W

_MINVAL = np.float32(1e-7)
_MAXVAL = np.float32(1.0 - 1e-7)
_SCALE = np.float32(_MAXVAL - _MINVAL)
_KS0 = 0
_KS1 = 42
_KS2 = 42 ^ 0x1BD11BDA
_ROT = (13, 15, 26, 6, 17, 29, 16, 24)


def _rotl(x, d):
    return lax.shift_left(x, jnp.int32(d)) | lax.shift_right_logical(
        x, jnp.int32(32 - d))


def _threefry_bits(flat):
    ks = (jnp.int32(_KS0), jnp.int32(_KS1), jnp.int32(_KS2))
    x1 = flat + ks[1]
    x0 = x1
    x1 = _rotl(x1, _ROT[0]) ^ x0
    for d in _ROT[1:4]:
        x0 = x0 + x1
        x1 = _rotl(x1, d) ^ x0
    x0 = x0 + ks[1]
    x1 = x1 + ks[2] + jnp.int32(1)
    for g in range(1, 5):
        rots = _ROT[0:4] if g % 2 == 0 else _ROT[4:8]
        for d in rots:
            x0 = x0 + x1
            x1 = _rotl(x1, d) ^ x0
        x0 = x0 + ks[(g + 1) % 3]
        x1 = x1 + ks[(g + 2) % 3] + jnp.int32(g + 1)
    return x0 ^ x1


def _gumbel_from_flat(flat):
    bits = _threefry_bits(flat)
    fb = lax.shift_right_logical(bits, jnp.int32(9)) | jnp.int32(0x3F800000)
    fl = lax.bitcast_convert_type(fb, jnp.float32) - jnp.float32(1.0)
    u = jnp.maximum(_MINVAL, fl * _SCALE + _MINVAL)
    return -jnp.log(-jnp.log(u))


def _kernel(x_ref, o_ref, rm_ref, ri_ref):
    j = pl.program_id(0)

    @pl.when(j == 0)
    def _init():
        rm_ref[:] = jnp.full((R, 1), -jnp.inf, jnp.float32)
        ri_ref[:] = jnp.zeros((R, 1), jnp.int32)

    base = j * W
    nsub = CHUNK // 128
    row = lax.broadcasted_iota(jnp.int32, (R, CHUNK), 0)
    lane = lax.broadcasted_iota(jnp.int32, (R, CHUNK), 1)
    lane128 = lax.broadcasted_iota(jnp.int32, (R, 128), 1)

    def body(t, carry):
        rm, ri = carry
        off = pl.multiple_of(t * CHUNK, CHUNK)
        colbase = base + off
        flat = row * C + (colbase + lane)
        g = _gumbel_from_flat(flat)
        v = x_ref[:, pl.ds(off, CHUNK)] + g
        vs = [v[:, k * 128:(k + 1) * 128] for k in range(nsub)]
        cs = [colbase + k * 128 + lane128 for k in range(nsub)]
        vs = [jnp.where(c < C, vv, -jnp.inf) for vv, c in zip(vs, cs)]
        while len(vs) > 1:
            nvs, ncs = [], []
            for a in range(0, len(vs) - 1, 2):
                keep = vs[a] >= vs[a + 1]
                nvs.append(jnp.where(keep, vs[a], vs[a + 1]))
                ncs.append(jnp.where(keep, cs[a], cs[a + 1]))
            if len(vs) % 2:
                nvs.append(vs[-1])
                ncs.append(cs[-1])
            vs, cs = nvs, ncs
        take = vs[0] > rm
        rm = jnp.where(take, vs[0], rm)
        ri = jnp.where(take, cs[0], ri)
        return rm, ri

    rm0 = jnp.full((R, 128), -jnp.inf, jnp.float32)
    ri0 = jnp.zeros((R, 128), jnp.int32)
    rm, ri = lax.fori_loop(0, W // CHUNK, body, (rm0, ri0), unroll=2)

    bm = jnp.max(rm, axis=1, keepdims=True)
    bi = jnp.min(jnp.where(rm == bm, ri, jnp.int32(0x7FFFFFFF)),
                 axis=1, keepdims=True)

    take = bm > rm_ref[:]
    rm_ref[:] = jnp.where(take, bm, rm_ref[:])
    ri_ref[:] = jnp.where(take, bi, ri_ref[:])

    @pl.when(j == NB - 1)
    def _fin():
        o_ref[:] = ri_ref[:]


@jax.jit
def kernel(logits):
    out = pl.pallas_call(
        _kernel,
        grid=(NB,),
        in_specs=[pl.BlockSpec((R, W), lambda j: (0, j))],
        out_specs=pl.BlockSpec((R, 1), lambda j: (0, 0)),
        out_shape=jax.ShapeDtypeStruct((R, 1), jnp.int32),
        scratch_shapes=[
            pltpu.VMEM((R, 1), jnp.float32),
            pltpu.VMEM((R, 1), jnp.int32),
        ],
    )(logits)
    return out.reshape(R)


# speedup vs baseline: 6.0760x; 1.0029x over previous
import functools

import jax
import jax.numpy as jnp
import numpy as np
from jax import lax
from jax.experimental import pallas as pl
from jax.experimental.pallas import tpu as pltpu

R = 64
C = 1_000_000
W = 12288
CHUNK = 384
NB = (C + W - 1) // W

_MINVAL = np.float32(1e-7)
_MAXVAL = np.float32(1.0 - 1e-7)
_SCALE = np.float32(_MAXVAL - _MINVAL)
_KS0 = 0
_KS1 = 42
_KS2 = 42 ^ 0x1BD11BDA
_ROT = (13, 15, 26, 6, 17, 29, 16, 24)


def _rotl(x, d):
    return lax.shift_left(x, jnp.int32(d)) | lax.shift_right_logical(
        x, jnp.int32(32 - d))


def _threefry_bits(flat):
    ks = (jnp.int32(_KS0), jnp.int32(_KS1), jnp.int32(_KS2))
    x1 = flat + ks[1]
    x0 = x1
    x1 = _rotl(x1, _ROT[0]) ^ x0
    for d in _ROT[1:4]:
        x0 = x0 + x1
        x1 = _rotl(x1, d) ^ x0
    x0 = x0 + ks[1]
    x1 = x1 + ks[2] + jnp.int32(1)
    for g in range(1, 5):
        rots = _ROT[0:4] if g % 2 == 0 else _ROT[4:8]
        for d in rots:
            x0 = x0 + x1
            x1 = _rotl(x1, d) ^ x0
        x0 = x0 + ks[(g + 1) % 3]
        x1 = x1 + ks[(g + 2) % 3] + jnp.int32(g + 1)
    return x0 ^ x1


def _gumbel_from_flat(flat):
    bits = _threefry_bits(flat)
    fb = lax.shift_right_logical(bits, jnp.int32(9)) | jnp.int32(0x3F800000)
    fl = lax.bitcast_convert_type(fb, jnp.float32) - jnp.float32(1.0)
    u = jnp.maximum(_MINVAL, fl * _SCALE + _MINVAL)
    return -jnp.log(-jnp.log(u))


def _kernel(x_ref, o_ref, rm_ref, ri_ref):
    j = pl.program_id(0)

    @pl.when(j == 0)
    def _init():
        rm_ref[:] = jnp.full((R, 1), -jnp.inf, jnp.float32)
        ri_ref[:] = jnp.zeros((R, 1), jnp.int32)

    base = j * W
    nsub = CHUNK // 128
    row = lax.broadcasted_iota(jnp.int32, (R, CHUNK), 0)
    lane = lax.broadcasted_iota(jnp.int32, (R, CHUNK), 1)
    lane128 = lax.broadcasted_iota(jnp.int32, (R, 128), 1)

    def body(t, carry):
        rm, ri = carry
        off = pl.multiple_of(t * CHUNK, CHUNK)
        colbase = base + off
        flat = row * C + (colbase + lane)
        g = _gumbel_from_flat(flat)
        v = x_ref[:, pl.ds(off, CHUNK)] + g
        vs = [v[:, k * 128:(k + 1) * 128] for k in range(nsub)]
        cs = [colbase + k * 128 + lane128 for k in range(nsub)]
        vs = [jnp.where(c < C, vv, -jnp.inf) for vv, c in zip(vs, cs)]
        while len(vs) > 1:
            nvs, ncs = [], []
            for a in range(0, len(vs) - 1, 2):
                keep = vs[a] >= vs[a + 1]
                nvs.append(jnp.where(keep, vs[a], vs[a + 1]))
                ncs.append(jnp.where(keep, cs[a], cs[a + 1]))
            if len(vs) % 2:
                nvs.append(vs[-1])
                ncs.append(cs[-1])
            vs, cs = nvs, ncs
        take = vs[0] > rm
        rm = jnp.where(take, vs[0], rm)
        ri = jnp.where(take, cs[0], ri)
        return rm, ri

    rm0 = jnp.full((R, 128), -jnp.inf, jnp.float32)
    ri0 = jnp.zeros((R, 128), jnp.int32)
    rm, ri = lax.fori_loop(0, W // CHUNK, body, (rm0, ri0), unroll=4)

    bm = jnp.max(rm, axis=1, keepdims=True)
    bi = jnp.min(jnp.where(rm == bm, ri, jnp.int32(0x7FFFFFFF)),
                 axis=1, keepdims=True)

    take = bm > rm_ref[:]
    rm_ref[:] = jnp.where(take, bm, rm_ref[:])
    ri_ref[:] = jnp.where(take, bi, ri_ref[:])

    @pl.when(j == NB - 1)
    def _fin():
        o_ref[:] = ri_ref[:]


@jax.jit
def kernel(logits):
    out = pl.pallas_call(
        _kernel,
        grid=(NB,),
        in_specs=[pl.BlockSpec((R, W), lambda j: (0, j))],
        out_specs=pl.BlockSpec((R, 1), lambda j: (0, 0)),
        out_shape=jax.ShapeDtypeStruct((R, 1), jnp.int32),
        scratch_shapes=[
            pltpu.VMEM((R, 1), jnp.float32),
            pltpu.VMEM((R, 1), jnp.int32),
        ],
    )(logits)
    return out.reshape(R)


# speedup vs baseline: 6.0970x; 1.0034x over previous
import functools

import jax
import jax.numpy as jnp
import numpy as np
from jax import lax
from jax.experimental import pallas as pl
from jax.experimental.pallas import tpu as pltpu

R = 64
C = 1_000_000
W = 12288
CHUNK = 384
NB = (C + W - 1) // W

_MINVAL = np.float32(1e-7)
_MAXVAL = np.float32(1.0 - 1e-7)
_SCALE = np.float32(_MAXVAL - _MINVAL)
_KS0 = 0
_KS1 = 42
_KS2 = 42 ^ 0x1BD11BDA
_ROT = (13, 15, 26, 6, 17, 29, 16, 24)


def _rotl(x, d):
    return lax.shift_left(x, jnp.int32(d)) | lax.shift_right_logical(
        x, jnp.int32(32 - d))


def _threefry_bits(flat):
    ks = (jnp.int32(_KS0), jnp.int32(_KS1), jnp.int32(_KS2))
    x1 = flat + ks[1]
    x0 = x1
    x1 = _rotl(x1, _ROT[0]) ^ x0
    for d in _ROT[1:4]:
        x0 = x0 + x1
        x1 = _rotl(x1, d) ^ x0
    x0 = x0 + ks[1]
    x1 = x1 + ks[2] + jnp.int32(1)
    for g in range(1, 5):
        rots = _ROT[0:4] if g % 2 == 0 else _ROT[4:8]
        for d in rots:
            x0 = x0 + x1
            x1 = _rotl(x1, d) ^ x0
        x0 = x0 + ks[(g + 1) % 3]
        x1 = x1 + ks[(g + 2) % 3] + jnp.int32(g + 1)
    return x0 ^ x1


def _gumbel_from_flat(flat):
    bits = _threefry_bits(flat)
    fb = lax.shift_right_logical(bits, jnp.int32(9)) | jnp.int32(0x3F800000)
    fl = lax.bitcast_convert_type(fb, jnp.float32) - jnp.float32(1.0)
    u = jnp.maximum(_MINVAL, fl * _SCALE + _MINVAL)
    return -jnp.log(-jnp.log(u))


def _kernel(x_ref, o_ref, rm_ref, ri_ref):
    j = pl.program_id(0)

    @pl.when(j == 0)
    def _init():
        rm_ref[:] = jnp.full((R, 1), -jnp.inf, jnp.float32)
        ri_ref[:] = jnp.zeros((R, 1), jnp.int32)

    base = j * W
    nsub = CHUNK // 128
    row = lax.broadcasted_iota(jnp.int32, (R, CHUNK), 0)
    lane = lax.broadcasted_iota(jnp.int32, (R, CHUNK), 1)
    lane128 = lax.broadcasted_iota(jnp.int32, (R, 128), 1)

    def body(t, carry):
        rm, ri = carry
        off = pl.multiple_of(t * CHUNK, CHUNK)
        colbase = base + off
        flat = row * C + (colbase + lane)
        g = _gumbel_from_flat(flat)
        v = x_ref[:, pl.ds(off, CHUNK)] + g
        vs = [v[:, k * 128:(k + 1) * 128] for k in range(nsub)]
        cs = [colbase + k * 128 + lane128 for k in range(nsub)]
        vs = [jnp.where(c < C, vv, -jnp.inf) for vv, c in zip(vs, cs)]
        while len(vs) > 1:
            nvs, ncs = [], []
            for a in range(0, len(vs) - 1, 2):
                keep = vs[a] >= vs[a + 1]
                nvs.append(jnp.where(keep, vs[a], vs[a + 1]))
                ncs.append(jnp.where(keep, cs[a], cs[a + 1]))
            if len(vs) % 2:
                nvs.append(vs[-1])
                ncs.append(cs[-1])
            vs, cs = nvs, ncs
        take = vs[0] > rm
        rm = jnp.where(take, vs[0], rm)
        ri = jnp.where(take, cs[0], ri)
        return rm, ri

    rm0 = jnp.full((R, 128), -jnp.inf, jnp.float32)
    ri0 = jnp.zeros((R, 128), jnp.int32)
    rm, ri = lax.fori_loop(0, W // CHUNK, body, (rm0, ri0), unroll=8)

    bm = jnp.max(rm, axis=1, keepdims=True)
    bi = jnp.min(jnp.where(rm == bm, ri, jnp.int32(0x7FFFFFFF)),
                 axis=1, keepdims=True)

    take = bm > rm_ref[:]
    rm_ref[:] = jnp.where(take, bm, rm_ref[:])
    ri_ref[:] = jnp.where(take, bi, ri_ref[:])

    @pl.when(j == NB - 1)
    def _fin():
        o_ref[:] = ri_ref[:]


@jax.jit
def kernel(logits):
    out = pl.pallas_call(
        _kernel,
        grid=(NB,),
        in_specs=[pl.BlockSpec((R, W), lambda j: (0, j))],
        out_specs=pl.BlockSpec((R, 1), lambda j: (0, 0)),
        out_shape=jax.ShapeDtypeStruct((R, 1), jnp.int32),
        scratch_shapes=[
            pltpu.VMEM((R, 1), jnp.float32),
            pltpu.VMEM((R, 1), jnp.int32),
        ],
    )(logits)
    return out.reshape(R)


# speedup vs baseline: 6.1027x; 1.0009x over previous
import functools

import jax
import jax.numpy as jnp
import numpy as np
from jax import lax
from jax.experimental import pallas as pl
from jax.experimental.pallas import tpu as pltpu

R = 64
C = 1_000_000
W = 12288
CHUNK = 384
NB = (C + W - 1) // W

_MINVAL = np.float32(1e-7)
_MAXVAL = np.float32(1.0 - 1e-7)
_SCALE = np.float32(_MAXVAL - _MINVAL)
_KS0 = 0
_KS1 = 42
_KS2 = 42 ^ 0x1BD11BDA
_ROT = (13, 15, 26, 6, 17, 29, 16, 24)


def _rotl(x, d):
    return lax.shift_left(x, jnp.int32(d)) | lax.shift_right_logical(
        x, jnp.int32(32 - d))


def _threefry_bits(flat):
    ks = (jnp.int32(_KS0), jnp.int32(_KS1), jnp.int32(_KS2))
    x1 = flat + ks[1]
    x0 = x1
    x1 = _rotl(x1, _ROT[0]) ^ x0
    for d in _ROT[1:4]:
        x0 = x0 + x1
        x1 = _rotl(x1, d) ^ x0
    x0 = x0 + ks[1]
    x1 = x1 + ks[2] + jnp.int32(1)
    for g in range(1, 5):
        rots = _ROT[0:4] if g % 2 == 0 else _ROT[4:8]
        for d in rots:
            x0 = x0 + x1
            x1 = _rotl(x1, d) ^ x0
        x0 = x0 + ks[(g + 1) % 3]
        x1 = x1 + ks[(g + 2) % 3] + jnp.int32(g + 1)
    return x0 ^ x1


def _gumbel_from_flat(flat):
    bits = _threefry_bits(flat)
    fb = lax.shift_right_logical(bits, jnp.int32(9)) | jnp.int32(0x3F800000)
    fl = lax.bitcast_convert_type(fb, jnp.float32) - jnp.float32(1.0)
    u = jnp.maximum(_MINVAL, fl * _SCALE + _MINVAL)
    return -jnp.log(-jnp.log(u))


def _kernel(x_ref, o_ref, rm_ref, ri_ref):
    j = pl.program_id(0)

    @pl.when(j == 0)
    def _init():
        rm_ref[:] = jnp.full((R, 1), -jnp.inf, jnp.float32)
        ri_ref[:] = jnp.zeros((R, 1), jnp.int32)

    base = j * W
    nsub = CHUNK // 128
    row = lax.broadcasted_iota(jnp.int32, (R, CHUNK), 0)
    lane = lax.broadcasted_iota(jnp.int32, (R, CHUNK), 1)
    lane128 = lax.broadcasted_iota(jnp.int32, (R, 128), 1)

    def body(t, carry):
        rm, ri = carry
        off = pl.multiple_of(t * CHUNK, CHUNK)
        colbase = base + off
        flat = row * C + (colbase + lane)
        g = _gumbel_from_flat(flat)
        v = x_ref[:, pl.ds(off, CHUNK)] + g
        vs = [v[:, k * 128:(k + 1) * 128] for k in range(nsub)]
        cs = [colbase + k * 128 + lane128 for k in range(nsub)]
        vs = [jnp.where(c < C, vv, -jnp.inf) for vv, c in zip(vs, cs)]
        while len(vs) > 1:
            nvs, ncs = [], []
            for a in range(0, len(vs) - 1, 2):
                keep = vs[a] >= vs[a + 1]
                nvs.append(jnp.where(keep, vs[a], vs[a + 1]))
                ncs.append(jnp.where(keep, cs[a], cs[a + 1]))
            if len(vs) % 2:
                nvs.append(vs[-1])
                ncs.append(cs[-1])
            vs, cs = nvs, ncs
        take = vs[0] > rm
        rm = jnp.where(take, vs[0], rm)
        ri = jnp.where(take, cs[0], ri)
        return rm, ri

    rm0 = jnp.full((R, 128), -jnp.inf, jnp.float32)
    ri0 = jnp.zeros((R, 128), jnp.int32)
    rm, ri = lax.fori_loop(0, W // CHUNK, body, (rm0, ri0), unroll=16)

    bm = jnp.max(rm, axis=1, keepdims=True)
    bi = jnp.min(jnp.where(rm == bm, ri, jnp.int32(0x7FFFFFFF)),
                 axis=1, keepdims=True)

    take = bm > rm_ref[:]
    rm_ref[:] = jnp.where(take, bm, rm_ref[:])
    ri_ref[:] = jnp.where(take, bi, ri_ref[:])

    @pl.when(j == NB - 1)
    def _fin():
        o_ref[:] = ri_ref[:]


@jax.jit
def kernel(logits):
    out = pl.pallas_call(
        _kernel,
        grid=(NB,),
        in_specs=[pl.BlockSpec((R, W), lambda j: (0, j))],
        out_specs=pl.BlockSpec((R, 1), lambda j: (0, 0)),
        out_shape=jax.ShapeDtypeStruct((R, 1), jnp.int32),
        scratch_shapes=[
            pltpu.VMEM((R, 1), jnp.float32),
            pltpu.VMEM((R, 1), jnp.int32),
        ],
    )(logits)
    return out.reshape(R)


# speedup vs baseline: 6.1413x; 1.0063x over previous
import functools

import jax
import jax.numpy as jnp
import numpy as np
from jax import lax
from jax.experimental import pallas as pl
from jax.experimental.pallas import tpu as pltpu

R = 64
C = 1_000_000
W = 12288
CHUNK = 384
NB = (C + W - 1) // W

_MINVAL = np.float32(1e-7)
_MAXVAL = np.float32(1.0 - 1e-7)
_SCALE = np.float32(_MAXVAL - _MINVAL)
_KS0 = 0
_KS1 = 42
_KS2 = 42 ^ 0x1BD11BDA
_ROT = (13, 15, 26, 6, 17, 29, 16, 24)


def _rotl(x, d):
    return lax.shift_left(x, jnp.int32(d)) | lax.shift_right_logical(
        x, jnp.int32(32 - d))


def _threefry_bits(x1):
    ks = (jnp.int32(_KS0), jnp.int32(_KS1), jnp.int32(_KS2))
    x0 = x1
    x1 = _rotl(x1, _ROT[0]) ^ x0
    for d in _ROT[1:4]:
        x0 = x0 + x1
        x1 = _rotl(x1, d) ^ x0
    x0 = x0 + ks[1]
    x1 = x1 + ks[2] + jnp.int32(1)
    for g in range(1, 5):
        rots = _ROT[0:4] if g % 2 == 0 else _ROT[4:8]
        for d in rots:
            x0 = x0 + x1
            x1 = _rotl(x1, d) ^ x0
        x0 = x0 + ks[(g + 1) % 3]
        x1 = x1 + ks[(g + 2) % 3] + jnp.int32(g + 1)
    return x0 ^ x1


def _gumbel_from_x1(x1):
    bits = _threefry_bits(x1)
    fb = lax.shift_right_logical(bits, jnp.int32(9)) | jnp.int32(0x3F800000)
    fl = lax.bitcast_convert_type(fb, jnp.float32) - jnp.float32(1.0)
    u = jnp.maximum(_MINVAL, fl * _SCALE + _MINVAL)
    return -jnp.log(-jnp.log(u))


def _kernel(x_ref, o_ref, rm_ref, ri_ref):
    j = pl.program_id(0)

    @pl.when(j == 0)
    def _init():
        rm_ref[:] = jnp.full((R, 1), -jnp.inf, jnp.float32)
        ri_ref[:] = jnp.zeros((R, 1), jnp.int32)

    _PAD0 = C - (NB - 1) * W
    @pl.when(j == NB - 1)
    def _mask_tail():
        x_ref[:, pl.ds(_PAD0, W - _PAD0)] = jnp.full(
            (R, W - _PAD0), -jnp.inf, jnp.float32)

    base = j * W
    nsub = CHUNK // 128
    row = lax.broadcasted_iota(jnp.int32, (R, CHUNK), 0)
    lane = lax.broadcasted_iota(jnp.int32, (R, CHUNK), 1)
    lane128 = lax.broadcasted_iota(jnp.int32, (R, 128), 1)
    rowlane = row * jnp.int32(C) + lane

    def body(t, carry):
        rm, ri = carry
        off = pl.multiple_of(t * CHUNK, CHUNK)
        colbase = base + off
        g = _gumbel_from_x1(rowlane + (colbase + jnp.int32(_KS1)))
        v = x_ref[:, pl.ds(off, CHUNK)] + g
        vs = [v[:, k * 128:(k + 1) * 128] for k in range(nsub)]
        cs = [colbase + k * 128 + lane128 for k in range(nsub)]
        while len(vs) > 1:
            nvs, ncs = [], []
            for a in range(0, len(vs) - 1, 2):
                keep = vs[a] >= vs[a + 1]
                nvs.append(jnp.where(keep, vs[a], vs[a + 1]))
                ncs.append(jnp.where(keep, cs[a], cs[a + 1]))
            if len(vs) % 2:
                nvs.append(vs[-1])
                ncs.append(cs[-1])
            vs, cs = nvs, ncs
        take = vs[0] > rm
        rm = jnp.where(take, vs[0], rm)
        ri = jnp.where(take, cs[0], ri)
        return rm, ri

    rm0 = jnp.full((R, 128), -jnp.inf, jnp.float32)
    ri0 = jnp.zeros((R, 128), jnp.int32)
    rm, ri = lax.fori_loop(0, W // CHUNK, body, (rm0, ri0), unroll=16)

    bm = jnp.max(rm, axis=1, keepdims=True)
    bi = jnp.min(jnp.where(rm == bm, ri, jnp.int32(0x7FFFFFFF)),
                 axis=1, keepdims=True)

    take = bm > rm_ref[:]
    rm_ref[:] = jnp.where(take, bm, rm_ref[:])
    ri_ref[:] = jnp.where(take, bi, ri_ref[:])

    @pl.when(j == NB - 1)
    def _fin():
        o_ref[:] = ri_ref[:]


@jax.jit
def kernel(logits):
    out = pl.pallas_call(
        _kernel,
        grid=(NB,),
        in_specs=[pl.BlockSpec((R, W), lambda j: (0, j))],
        out_specs=pl.BlockSpec((R, 1), lambda j: (0, 0)),
        out_shape=jax.ShapeDtypeStruct((R, 1), jnp.int32),
        scratch_shapes=[
            pltpu.VMEM((R, 1), jnp.float32),
            pltpu.VMEM((R, 1), jnp.int32),
        ],
    )(logits)
    return out.reshape(R)


# speedup vs baseline: 6.2047x; 1.0103x over previous
import functools

import jax
import jax.numpy as jnp
import numpy as np
from jax import lax
from jax.experimental import pallas as pl
from jax.experimental.pallas import tpu as pltpu

R = 64
C = 1_000_000
W = 12288
CHUNK = 384
NB = (C + W - 1) // W

_MINVAL = np.float32(1e-7)
_MAXVAL = np.float32(1.0 - 1e-7)
_SCALE = np.float32(_MAXVAL - _MINVAL)
_KS0 = 0
_KS1 = 42
_KS2 = 42 ^ 0x1BD11BDA
_ROT = (13, 15, 26, 6, 17, 29, 16, 24)


def _rotl(x, d):
    return lax.shift_left(x, jnp.int32(d)) | lax.shift_right_logical(
        x, jnp.int32(32 - d))


def _threefry_bits(x1):
    ks = (jnp.int32(_KS0), jnp.int32(_KS1), jnp.int32(_KS2))
    x0 = x1
    x1 = _rotl(x1, _ROT[0]) ^ x0
    for d in _ROT[1:4]:
        x0 = x0 + x1
        x1 = _rotl(x1, d) ^ x0
    x0 = x0 + ks[1]
    x1 = x1 + ks[2] + jnp.int32(1)
    for g in range(1, 5):
        rots = _ROT[0:4] if g % 2 == 0 else _ROT[4:8]
        for d in rots:
            x0 = x0 + x1
            x1 = _rotl(x1, d) ^ x0
        x0 = x0 + ks[(g + 1) % 3]
        x1 = x1 + ks[(g + 2) % 3] + jnp.int32(g + 1)
    return x0 ^ x1


def _gumbel_from_x1(x1):
    bits = _threefry_bits(x1)
    fb = lax.shift_right_logical(bits, jnp.int32(9)) | jnp.int32(0x3F800000)
    fl = lax.bitcast_convert_type(fb, jnp.float32) - jnp.float32(1.0)
    u = jnp.maximum(_MINVAL, fl * _SCALE + _MINVAL)
    return -jnp.log(-jnp.log(u))


def _kernel(x_ref, o_ref, rm_ref, ri_ref):
    j = pl.program_id(0)

    @pl.when(j == 0)
    def _init():
        rm_ref[:] = jnp.full((R, 1), -jnp.inf, jnp.float32)
        ri_ref[:] = jnp.zeros((R, 1), jnp.int32)

    _PAD0 = C - (NB - 1) * W
    @pl.when(j == NB - 1)
    def _mask_tail():
        x_ref[:, pl.ds(_PAD0, W - _PAD0)] = jnp.full(
            (R, W - _PAD0), -jnp.inf, jnp.float32)

    base = j * W
    nsub = CHUNK // 128
    row = lax.broadcasted_iota(jnp.int32, (R, CHUNK), 0)
    lane = lax.broadcasted_iota(jnp.int32, (R, CHUNK), 1)
    lane128 = lax.broadcasted_iota(jnp.int32, (R, 128), 1)
    rowlane = row * jnp.int32(C) + lane

    def body(t, carry):
        rm, ri = carry
        off = pl.multiple_of(t * CHUNK, CHUNK)
        colbase = base + off
        g = _gumbel_from_x1(rowlane + (colbase + jnp.int32(_KS1)))
        v = x_ref[:, pl.ds(off, CHUNK)] + g
        vs = [v[:, k * 128:(k + 1) * 128] for k in range(nsub)]
        cs = [colbase + k * 128 + lane128 for k in range(nsub)]
        while len(vs) > 1:
            nvs, ncs = [], []
            for a in range(0, len(vs) - 1, 2):
                keep = vs[a] >= vs[a + 1]
                nvs.append(jnp.where(keep, vs[a], vs[a + 1]))
                ncs.append(jnp.where(keep, cs[a], cs[a + 1]))
            if len(vs) % 2:
                nvs.append(vs[-1])
                ncs.append(cs[-1])
            vs, cs = nvs, ncs
        take = vs[0] > rm
        rm = jnp.where(take, vs[0], rm)
        ri = jnp.where(take, cs[0], ri)
        return rm, ri

    rm0 = jnp.full((R, 128), -jnp.inf, jnp.float32)
    ri0 = jnp.zeros((R, 128), jnp.int32)
    rm, ri = lax.fori_loop(0, W // CHUNK, body, (rm0, ri0), unroll=32)

    bm = jnp.max(rm, axis=1, keepdims=True)
    bi = jnp.min(jnp.where(rm == bm, ri, jnp.int32(0x7FFFFFFF)),
                 axis=1, keepdims=True)

    take = bm > rm_ref[:]
    rm_ref[:] = jnp.where(take, bm, rm_ref[:])
    ri_ref[:] = jnp.where(take, bi, ri_ref[:])

    @pl.when(j == NB - 1)
    def _fin():
        o_ref[:] = ri_ref[:]


@jax.jit
def kernel(logits):
    out = pl.pallas_call(
        _kernel,
        grid=(NB,),
        in_specs=[pl.BlockSpec((R, W), lambda j: (0, j))],
        out_specs=pl.BlockSpec((R, 1), lambda j: (0, 0)),
        out_shape=jax.ShapeDtypeStruct((R, 1), jnp.int32),
        scratch_shapes=[
            pltpu.VMEM((R, 1), jnp.float32),
            pltpu.VMEM((R, 1), jnp.int32),
        ],
    )(logits)
    return out.reshape(R)
